# baseline (speedup 1.0000x reference)
import functools

import numpy as np
import jax
import jax.numpy as jnp
from jax import lax
from jax.experimental import pallas as pl
from jax.experimental.pallas import tpu as pltpu
from jax.experimental.pallas import tpu_sc as plsc

_D = 128
_MAX_LEN = 1000
_B = 1024
_L = 128
_NC = 2
_NS = 16
_NW = _NC * _NS
_N = _B * _L
_PER_W = _N // _NW
_CH = 128
_NCH = _PER_W // _CH
_SC_ROWS = 256
_NSC = _PER_W // _SC_ROWS
_NB = 2
_NG = _NSC // _NB


def _make_pe():
    position = np.arange(_MAX_LEN, dtype=np.float32)[:, None]
    div_term = np.exp(
        np.arange(0, _D, 2, dtype=np.float32) * (-np.log(10000.0) / _D))
    pe = np.zeros((_MAX_LEN, _D), dtype=np.float32)
    pe[:, 0::2] = np.sin(position * div_term)
    pe[:, 1::2] = np.cos(position * div_term)
    return jnp.asarray(pe[:_L])


_mesh = plsc.VectorSubcoreMesh(
    core_axis_name="c", subcore_axis_name="s",
    num_cores=_NC, num_subcores=_NS)


def _gather2(w_hbm, idx_v, sc_idx, buf, sem):
    pltpu.async_copy(
        w_hbm.at[idx_v.at[2 * sc_idx]], buf.at[pl.ds(0, _CH)], sem)
    pltpu.async_copy(
        w_hbm.at[idx_v.at[2 * sc_idx + 1]], buf.at[pl.ds(_CH, _CH)], sem)


def _wait2(w_hbm, idx_v, sc_idx, buf, sem):
    pltpu.make_async_copy(
        w_hbm.at[idx_v.at[2 * sc_idx]], buf.at[pl.ds(0, _CH)], sem).wait()
    pltpu.make_async_copy(
        w_hbm.at[idx_v.at[2 * sc_idx + 1]], buf.at[pl.ds(_CH, _CH)],
        sem).wait()


@functools.partial(
    pl.kernel,
    out_type=jax.ShapeDtypeStruct((_N, _D), jnp.float32),
    mesh=_mesh,
    scratch_types=(
        [pltpu.VMEM((_NCH, _CH), jnp.int32),
         pltpu.VMEM((_L, _D), jnp.float32)]
        + [pltpu.VMEM((_SC_ROWS, _D), jnp.float32) for _ in range(_NB)]
        + [pltpu.SemaphoreType.DMA for _ in range(2 * _NB)]
    ),
)
def _emb_kernel(w_hbm, xr_hbm, pe_hbm, out_hbm, idx_v, pe_v, *sc):
    bufs = sc[:_NB]
    gsems = sc[_NB:2 * _NB]
    osems = sc[2 * _NB:]
    wid = lax.axis_index("s") * _NC + lax.axis_index("c")
    base = wid * _PER_W
    pltpu.sync_copy(xr_hbm.at[wid], idx_v)
    pltpu.sync_copy(pe_hbm, pe_v)

    for b in range(_NB):
        _gather2(w_hbm, idx_v, b, bufs[b], gsems[b])

    def add_pe(buf):
        def rows(r2, c_):
            r = r2 * 2
            for rr in (r, r + 1):
                for c in range(8):
                    s = pl.ds(c * 16, 16)
                    buf[rr, s] = buf[rr, s] + pe_v[rr, s]
                    buf[rr + _L, s] = buf[rr + _L, s] + pe_v[rr, s]
            return c_
        lax.fori_loop(0, _L // 2, rows, 0)

    def group(g, carry):
        a0 = g * _NB
        nxt = a0 + _NB
        more = g < _NG - 1
        for b in range(_NB):
            _wait2(w_hbm, idx_v, a0 + b, bufs[b], gsems[b])
            add_pe(bufs[b])
            pltpu.async_copy(
                bufs[b],
                out_hbm.at[pl.ds(base + (a0 + b) * _SC_ROWS, _SC_ROWS)],
                osems[b])
            if b >= 1:
                @pl.when(more)
                def _(b=b):
                    pltpu.make_async_copy(
                        bufs[b - 1], out_hbm.at[pl.ds(base, _SC_ROWS)],
                        osems[b - 1]).wait()
                    _gather2(w_hbm, idx_v, nxt + b - 1, bufs[b - 1],
                             gsems[b - 1])

        @pl.when(more)
        def _():
            pltpu.make_async_copy(
                bufs[_NB - 1], out_hbm.at[pl.ds(base, _SC_ROWS)],
                osems[_NB - 1]).wait()
            _gather2(w_hbm, idx_v, nxt + _NB - 1, bufs[_NB - 1],
                     gsems[_NB - 1])
        return carry

    lax.fori_loop(0, _NG, group, 0)
    for b in range(_NB):
        pltpu.make_async_copy(
            bufs[b], out_hbm.at[pl.ds(base, _SC_ROWS)], osems[b]).wait()


def kernel(x, W):
    pe = _make_pe()
    xr = x.reshape(_NW, _NCH, _CH)
    out = _emb_kernel(W, xr, pe)
    return out.reshape(_B, _L, _D)


# speedup vs baseline: 1.1550x; 1.1550x over previous
import functools

import numpy as np
import jax
import jax.numpy as jnp
from jax import lax
from jax.experimental import pallas as pl
from jax.experimental.pallas import tpu as pltpu
from jax.experimental.pallas import tpu_sc as plsc

_D = 128
_MAX_LEN = 1000
_B = 1024
_L = 128
_NC = 2
_NS = 16
_NW = _NC * _NS
_N = _B * _L
_PER_W = _N // _NW
_CH = 128
_NCH = _PER_W // _CH
_NB = 4
_NG = _NCH // _NB


def _make_pe():
    position = np.arange(_MAX_LEN, dtype=np.float32)[:, None]
    div_term = np.exp(
        np.arange(0, _D, 2, dtype=np.float32) * (-np.log(10000.0) / _D))
    pe = np.zeros((_MAX_LEN, _D), dtype=np.float32)
    pe[:, 0::2] = np.sin(position * div_term)
    pe[:, 1::2] = np.cos(position * div_term)
    return jnp.asarray(pe[:_L])


_mesh = plsc.VectorSubcoreMesh(
    core_axis_name="c", subcore_axis_name="s",
    num_cores=_NC, num_subcores=_NS)


@functools.partial(
    pl.kernel,
    out_type=jax.ShapeDtypeStruct((_N, _D), jnp.float32),
    mesh=_mesh,
    scratch_types=(
        [pltpu.VMEM((_NCH, _CH), jnp.int32),
         pltpu.VMEM((_L, _D), jnp.float32)]
        + [pltpu.VMEM((_CH, _D), jnp.float32) for _ in range(_NB)]
        + [pltpu.SemaphoreType.DMA for _ in range(2 * _NB)]
    ),
)
def _emb_kernel(w_hbm, xr_hbm, pe_hbm, out_hbm, idx_v, pe_v, *sc):
    bufs = sc[:_NB]
    gsems = sc[_NB:2 * _NB]
    osems = sc[2 * _NB:]
    wid = lax.axis_index("s") * _NC + lax.axis_index("c")
    base = wid * _PER_W
    pltpu.sync_copy(xr_hbm.at[wid], idx_v)
    pltpu.sync_copy(pe_hbm, pe_v)

    for b in range(_NB):
        pltpu.async_copy(w_hbm.at[idx_v.at[b]], bufs[b], gsems[b])

    def add_pe(buf):
        def rows(r2, c_):
            r = r2 * 2
            for rr in (r, r + 1):
                for c in range(8):
                    s = pl.ds(c * 16, 16)
                    buf[rr, s] = buf[rr, s] + pe_v[rr, s]
            return c_
        lax.fori_loop(0, _CH // 2, rows, 0)

    def group(g, carry):
        a0 = g * _NB
        nxt = a0 + _NB
        more = g < _NG - 1
        for b in range(_NB):
            pltpu.make_async_copy(
                w_hbm.at[idx_v.at[a0 + b]], bufs[b], gsems[b]).wait()
            add_pe(bufs[b])
            pltpu.async_copy(
                bufs[b], out_hbm.at[pl.ds(base + (a0 + b) * _CH, _CH)],
                osems[b])
            if b >= 1:
                @pl.when(more)
                def _(b=b):
                    pltpu.make_async_copy(
                        bufs[b - 1], out_hbm.at[pl.ds(base, _CH)],
                        osems[b - 1]).wait()
                    pltpu.async_copy(
                        w_hbm.at[idx_v.at[nxt + b - 1]], bufs[b - 1],
                        gsems[b - 1])

        @pl.when(more)
        def _():
            pltpu.make_async_copy(
                bufs[_NB - 1], out_hbm.at[pl.ds(base, _CH)],
                osems[_NB - 1]).wait()
            pltpu.async_copy(
                w_hbm.at[idx_v.at[nxt + _NB - 1]], bufs[_NB - 1],
                gsems[_NB - 1])
        return carry

    lax.fori_loop(0, _NG, group, 0)
    for b in range(_NB):
        pltpu.make_async_copy(
            bufs[b], out_hbm.at[pl.ds(base, _CH)], osems[b]).wait()


def kernel(x, W):
    pe = _make_pe()
    xr = x.reshape(_NW, _NCH, _CH)
    out = _emb_kernel(W, xr, pe)
    return out.reshape(_B, _L, _D)
